# baseline (speedup 1.0000x reference)
import functools

import jax
import jax.numpy as jnp
import numpy as np
from jax import lax
from jax.experimental import pallas as pl
from jax.experimental.pallas import tpu as pltpu
from jax.experimental.pallas import tpu_sc as plsc

N_TOKENS = 8192
D_MODEL = 4096
N_PATHS = 16
NC = 2
NS = 16
NW = NC * NS
RPW = N_TOKENS // NW
CH = 4
NCHUNK = RPW // CH
NBUF = 7


@functools.lru_cache(maxsize=None)
def _routing_tables():
    with jax.ensure_compile_time_eval():
        score = np.asarray(jax.random.normal(
            jax.random.key(42), (N_TOKENS, N_PATHS), dtype=jnp.float32))
    route = np.argmax(score, axis=1).astype(np.int32)
    counts = np.bincount(route, minlength=N_PATHS).astype(np.int32)
    starts = np.zeros(N_PATHS, np.int32)
    starts[1:] = np.cumsum(counts)[:-1]
    rank = np.zeros(N_TOKENS, np.int32)
    cnt = np.zeros(N_PATHS, np.int64)
    for i, p in enumerate(route):
        rank[i] = cnt[p]
        cnt[p] += 1
    pos = (starts[route] + rank).astype(np.int32)
    order = np.zeros(N_TOKENS, np.int32)
    order[pos] = np.arange(N_TOKENS, dtype=np.int32)
    route_sorted = np.sort(route).astype(np.int32)
    return order.reshape(N_TOKENS // CH, CH), route_sorted, counts


def _dispatch_sc(x, pos2d, rs2d, cnts):
    mesh = plsc.VectorSubcoreMesh(core_axis_name="c", subcore_axis_name="s")

    @functools.partial(
        pl.kernel,
        out_type=[
            jax.ShapeDtypeStruct((N_TOKENS, D_MODEL), jnp.float32),
            jax.ShapeDtypeStruct((N_TOKENS // 16, 16), jnp.int32),
            jax.ShapeDtypeStruct((N_PATHS,), jnp.int32),
        ],
        mesh=mesh,
        scratch_types=(
            [pltpu.VMEM((NCHUNK, CH), jnp.int32)]
            + [pltpu.VMEM((16, 16), jnp.int32)]
            + [pltpu.VMEM((N_PATHS,), jnp.int32)]
            + [pltpu.VMEM((CH, D_MODEL), jnp.float32)] * NBUF
            + [pltpu.SemaphoreType.DMA] * (2 * NBUF + 1)
        ),
    )
    def dispatch(x_hbm, pos_hbm, rs_hbm, cnt_hbm,
                 out_hbm, rs_out, cnt_out, pos_v, rs_v, cnt_v, *bufs_and_sems):
        buf = bufs_and_sems[:NBUF]
        sin = bufs_and_sems[NBUF:2 * NBUF]
        sout = bufs_and_sems[2 * NBUF:3 * NBUF]
        saux = bufs_and_sems[3 * NBUF]
        wid = lax.axis_index("s") * NC + lax.axis_index("c")
        base = wid * RPW

        pltpu.sync_copy(pos_hbm.at[pl.ds(wid * NCHUNK, NCHUNK)], pos_v)
        pltpu.sync_copy(rs_hbm.at[pl.ds(wid * 16, 16)], rs_v)
        pltpu.async_copy(rs_v, rs_out.at[pl.ds(wid * 16, 16)], saux)

        @pl.when(wid == 0)
        def _():
            pltpu.sync_copy(cnt_hbm, cnt_v)
            pltpu.sync_copy(cnt_v, cnt_out)

        def start_in(k, b):
            pltpu.async_copy(x_hbm.at[pos_v.at[k]], buf[b], sin[b])

        def wait_in(k, b):
            pltpu.make_async_copy(
                x_hbm.at[pos_v.at[k]], buf[b], sin[b]).wait()

        def start_out(k, b):
            pltpu.async_copy(buf[b], out_hbm.at[pl.ds(base + k * CH, CH)], sout[b])

        def wait_out(k, b):
            pltpu.make_async_copy(
                buf[b], out_hbm.at[pl.ds(base + k * CH, CH)], sout[b]).wait()

        for k in range(NBUF - 1):
            start_in(k, k % NBUF)
        for k in range(NCHUNK):
            b = k % NBUF
            p = k + NBUF - 1
            if p < NCHUNK:
                bp = p % NBUF
                if k >= 1:
                    wait_out(k - 1, bp)
                start_in(p, bp)
            wait_in(k, b)
            start_out(k, b)
        for k in range(NCHUNK - NBUF, NCHUNK):
            if k >= 0:
                wait_out(k, k % NBUF)
        pltpu.make_async_copy(rs_v, rs_out.at[pl.ds(wid * 16, 16)], saux).wait()

    return dispatch(x, pos2d, rs2d, cnts)


def kernel(inputs):
    order2d, route_sorted, counts = _routing_tables()
    dispatched, rs, cnt = _dispatch_sc(
        inputs, jnp.asarray(order2d),
        jnp.asarray(route_sorted.reshape(N_TOKENS // 16, 16)),
        jnp.asarray(counts))
    return dispatched, rs.reshape(N_TOKENS), cnt


# speedup vs baseline: 1.0128x; 1.0128x over previous
import functools

import jax
import jax.numpy as jnp
import numpy as np
from jax import lax
from jax.experimental import pallas as pl
from jax.experimental.pallas import tpu as pltpu
from jax.experimental.pallas import tpu_sc as plsc

N_TOKENS = 8192
D_MODEL = 4096
N_PATHS = 16
NC = 2
NS = 16
NW = NC * NS
RPW = N_TOKENS // NW
CH = 4
NCHUNK = RPW // CH
NBUF = 7


@functools.lru_cache(maxsize=None)
def _routing_tables():
    with jax.ensure_compile_time_eval():
        score = np.asarray(jax.random.normal(
            jax.random.key(42), (N_TOKENS, N_PATHS), dtype=jnp.float32))
    route = np.argmax(score, axis=1).astype(np.int32)
    counts = np.bincount(route, minlength=N_PATHS).astype(np.int32)
    starts = np.zeros(N_PATHS, np.int32)
    starts[1:] = np.cumsum(counts)[:-1]
    rank = np.zeros(N_TOKENS, np.int32)
    cnt = np.zeros(N_PATHS, np.int64)
    for i, p in enumerate(route):
        rank[i] = cnt[p]
        cnt[p] += 1
    pos = (starts[route] + rank).astype(np.int32)
    order = np.zeros(N_TOKENS, np.int32)
    order[pos] = np.arange(N_TOKENS, dtype=np.int32)
    route_sorted = np.sort(route).astype(np.int32)
    return order.reshape(N_TOKENS // CH, CH), route_sorted, counts


def _dispatch_sc(x, pos2d):
    mesh = plsc.VectorSubcoreMesh(core_axis_name="c", subcore_axis_name="s")

    @functools.partial(
        pl.kernel,
        out_type=jax.ShapeDtypeStruct((N_TOKENS, D_MODEL), jnp.float32),
        mesh=mesh,
        scratch_types=(
            [pltpu.VMEM((NCHUNK, CH), jnp.int32)]
            + [pltpu.VMEM((CH, D_MODEL), jnp.float32)] * NBUF
            + [pltpu.SemaphoreType.DMA] * (2 * NBUF)
        ),
    )
    def dispatch(x_hbm, pos_hbm, out_hbm, pos_v, *bufs_and_sems):
        buf = bufs_and_sems[:NBUF]
        sin = bufs_and_sems[NBUF:2 * NBUF]
        sout = bufs_and_sems[2 * NBUF:3 * NBUF]
        wid = lax.axis_index("s") * NC + lax.axis_index("c")
        base = wid * RPW

        pltpu.sync_copy(pos_hbm.at[pl.ds(wid * NCHUNK, NCHUNK)], pos_v)

        def start_in(k, b):
            pltpu.async_copy(x_hbm.at[pos_v.at[k]], buf[b], sin[b])

        def wait_in(k, b):
            pltpu.make_async_copy(
                x_hbm.at[pos_v.at[k]], buf[b], sin[b]).wait()

        def start_out(k, b):
            pltpu.async_copy(buf[b], out_hbm.at[pl.ds(base + k * CH, CH)], sout[b])

        def wait_out(k, b):
            pltpu.make_async_copy(
                buf[b], out_hbm.at[pl.ds(base + k * CH, CH)], sout[b]).wait()

        for k in range(NBUF - 1):
            start_in(k, k % NBUF)
        for k in range(NCHUNK):
            b = k % NBUF
            p = k + NBUF - 1
            if p < NCHUNK:
                bp = p % NBUF
                if k >= 1:
                    wait_out(k - 1, bp)
                start_in(p, bp)
            wait_in(k, b)
            start_out(k, b)
        for k in range(NCHUNK - NBUF, NCHUNK):
            if k >= 0:
                wait_out(k, k % NBUF)

    return dispatch(x, pos2d)


def kernel(inputs):
    order2d, route_sorted, counts = _routing_tables()
    dispatched = _dispatch_sc(inputs, jnp.asarray(order2d))
    return dispatched, jnp.asarray(route_sorted), jnp.asarray(counts)
